# baseline (speedup 1.0000x reference)
import functools

import jax
import jax.numpy as jnp
from jax.experimental import pallas as pl
from jax.experimental.pallas import tpu as pltpu


def _fused_kernel(x_ref, w1_ref, b1_ref, s1_ref, w2_ref, b2_ref, g_ref,
                  beta_ref, o_ref, pad1_ref, z1_ref, *,
                  lth, H, W, Cin, C_tot, C):
    t = pl.program_id(1)
    nt = pl.num_programs(1)
    r0 = pl.multiple_of(t * lth, lth)
    HH = 2 * lth
    WH = 2 * W

    pad1_ref[...] = jnp.zeros_like(pad1_ref)
    pad1_ref[2:lth + 2, 1:W + 1, :] = (
        x_ref[0, pl.ds(r0, lth), :, :].astype(jnp.bfloat16))

    @pl.when(t > 0)
    def _():
        top = jnp.maximum(r0 - 2, 0)
        pad1_ref[0:2, 1:W + 1, :] = (
            x_ref[0, pl.ds(top, 2), :, :].astype(jnp.bfloat16))

    @pl.when(t < nt - 1)
    def _():
        bot = jnp.minimum(r0 + lth, H - 2)
        pad1_ref[lth + 2:lth + 4, 1:W + 1, :] = (
            x_ref[0, pl.ds(bot, 2), :, :].astype(jnp.bfloat16))

    M1 = lth + 2
    taps1 = [pad1_ref[dy:dy + M1, dx:dx + W, :]
             for dy in range(3) for dx in range(3)]
    patches1 = jnp.concatenate(taps1, axis=-1).reshape(M1 * W, 9 * Cin)

    y1 = jnp.dot(patches1, w1_ref[...],
                 preferred_element_type=jnp.float32) + b1_ref[...]
    y1 = jnp.where(y1 < 0, y1 * s1_ref[...], y1)

    z1_ref[:, 1:W + 1, :] = y1.reshape(M1, W, C_tot).astype(jnp.bfloat16)
    z1_ref[:, 0:1, :] = jnp.zeros((M1, 1, C_tot), jnp.bfloat16)
    z1_ref[:, W + 1:W + 2, :] = jnp.zeros((M1, 1, C_tot), jnp.bfloat16)

    @pl.when(t == 0)
    def _():
        z1_ref[0:1, :, :] = jnp.zeros((1, W + 2, C_tot), jnp.bfloat16)

    @pl.when(t == nt - 1)
    def _():
        z1_ref[M1 - 1:M1, :, :] = jnp.zeros((1, W + 2, C_tot), jnp.bfloat16)

    for a in range(2):
        for b in range(2):
            taps2 = []
            for dy in range(-1, 2):
                ap, sy = (a + dy) & 1, (a + dy) >> 1
                for dx in range(-1, 2):
                    bp, sx = (b + dx) & 1, (b + dx) >> 1
                    pp = 2 * ap + bp
                    taps2.append(
                        z1_ref[1 + sy:1 + sy + lth,
                               1 + sx:1 + sx + W,
                               pp * 128:pp * 128 + 64])
            patches2 = jnp.concatenate(taps2, axis=-1).reshape(lth * W, 9 * C)

            y = jnp.dot(patches2, w2_ref[...],
                        preferred_element_type=jnp.float32) + b2_ref[...]
            norm = jnp.dot((y * y).astype(jnp.bfloat16), g_ref[...],
                           preferred_element_type=jnp.float32) + beta_ref[...]
            p = 2 * a + b
            ident = z1_ref[1:1 + lth, 1:W + 1,
                           p * 128 + 64:p * 128 + 128].astype(jnp.float32)
            out = y * jnp.sqrt(norm) + ident.reshape(lth * W, C)

            o_ref[0, a:HH:2, b:WH:2, :] = out.reshape(lth, W, C)


def kernel(x_nhwc, subpel_w, subpel_b, conv_w, conv_b, gamma, beta, up_w, up_b):
    N, H, W, Cin = x_nhwc.shape
    C_main = subpel_w.shape[-1]
    C_tot = 2 * C_main
    C = conv_w.shape[-1]
    r = 2

    c = jnp.arange(C_main // (r * r))
    perm = jnp.concatenate(
        [jnp.concatenate([c * 4 + p, C_main + c * 4 + p]) for p in range(4)])
    w_cat = jnp.concatenate([subpel_w.reshape(9 * Cin, C_main),
                             up_w.reshape(9 * Cin, C_main)], axis=1)
    w_cat = w_cat[:, perm].astype(jnp.bfloat16)
    b_cat = jnp.concatenate([subpel_b, up_b])[perm].reshape(1, C_tot)
    slope = jnp.where(perm < C_main, 0.01, 1.0).reshape(1, C_tot)

    w2 = conv_w.reshape(9 * C, C).astype(jnp.bfloat16)
    b2 = conv_b.reshape(1, C)
    g2 = gamma.astype(jnp.bfloat16)
    beta2 = beta.reshape(1, C)

    lth = 32 if H % 32 == 0 else H
    nt = H // lth
    HH = 2 * lth
    body = functools.partial(_fused_kernel, lth=lth, H=H, W=W, Cin=Cin,
                             C_tot=C_tot, C=C)
    out = pl.pallas_call(
        body,
        out_shape=jax.ShapeDtypeStruct((N, r * H, r * W, C), jnp.float32),
        grid=(N, nt),
        in_specs=[
            pl.BlockSpec((1, H, W, Cin), lambda n, t: (n, 0, 0, 0)),
            pl.BlockSpec((9 * Cin, C_tot), lambda n, t: (0, 0)),
            pl.BlockSpec((1, C_tot), lambda n, t: (0, 0)),
            pl.BlockSpec((1, C_tot), lambda n, t: (0, 0)),
            pl.BlockSpec((9 * C, C), lambda n, t: (0, 0)),
            pl.BlockSpec((1, C), lambda n, t: (0, 0)),
            pl.BlockSpec((C, C), lambda n, t: (0, 0)),
            pl.BlockSpec((1, C), lambda n, t: (0, 0)),
        ],
        out_specs=pl.BlockSpec((1, HH, r * W, C), lambda n, t: (n, t, 0, 0)),
        scratch_shapes=[
            pltpu.VMEM((lth + 4, W + 2, Cin), jnp.bfloat16),
            pltpu.VMEM((lth + 2, W + 2, C_tot), jnp.bfloat16),
        ],
        compiler_params=pltpu.CompilerParams(
            dimension_semantics=("parallel", "parallel")),
    )(x_nhwc, w_cat, b_cat, slope, w2, b2, g2, beta2)

    return out


# speedup vs baseline: 1.1969x; 1.1969x over previous
import functools

import jax
import jax.numpy as jnp
from jax.experimental import pallas as pl
from jax.experimental.pallas import tpu as pltpu


def _fused_kernel(x_ref, w1_ref, b1_ref, s1_ref, w2_ref, b2_ref, g_ref,
                  beta_ref, o_ref, pad1_ref, pad2_ref, id_ref, *,
                  lth, H, W, Cin, C_tot, C):
    t = pl.program_id(1)
    nt = pl.num_programs(1)
    r0 = pl.multiple_of(t * lth, lth)
    HH = 2 * lth
    WH = 2 * W

    pad1_ref[...] = jnp.zeros_like(pad1_ref)
    pad1_ref[2:lth + 2, 1:W + 1, :] = (
        x_ref[0, pl.ds(r0, lth), :, :].astype(jnp.bfloat16))

    @pl.when(t > 0)
    def _():
        top = jnp.maximum(r0 - 2, 0)
        pad1_ref[0:2, 1:W + 1, :] = (
            x_ref[0, pl.ds(top, 2), :, :].astype(jnp.bfloat16))

    @pl.when(t < nt - 1)
    def _():
        bot = jnp.minimum(r0 + lth, H - 2)
        pad1_ref[lth + 2:lth + 4, 1:W + 1, :] = (
            x_ref[0, pl.ds(bot, 2), :, :].astype(jnp.bfloat16))

    M1 = lth + 2
    taps1 = [pad1_ref[dy:dy + M1, dx:dx + W, :]
             for dy in range(3) for dx in range(3)]
    patches1 = jnp.concatenate(taps1, axis=-1).reshape(M1 * W, 9 * Cin)

    y1 = jnp.dot(patches1, w1_ref[...],
                 preferred_element_type=jnp.float32) + b1_ref[...]

    y1 = jnp.where(y1 < 0, y1 * s1_ref[...], y1)
    y1 = y1.reshape(M1, W, C_tot)

    for b in range(2):
        pad2_ref[0:HH + 1:2, 1 + b:1 + WH:2, :] = (
            y1[0:lth + 1, :, (2 + b) * 128:(2 + b) * 128 + 64])
        pad2_ref[1:HH + 2:2, 1 + b:1 + WH:2, :] = (
            y1[1:lth + 2, :, b * 128:b * 128 + 64])
        id_ref[1:HH:2, b:WH:2, :] = (
            y1[1:lth + 1, :, (2 + b) * 128 + 64:(2 + b) * 128 + 128])
        id_ref[0:HH:2, b:WH:2, :] = (
            y1[1:lth + 1, :, b * 128 + 64:b * 128 + 128])

    pad2_ref[:, 0:1, :] = jnp.zeros((HH + 2, 1, C), jnp.float32)
    pad2_ref[:, WH + 1:WH + 2, :] = jnp.zeros((HH + 2, 1, C), jnp.float32)

    @pl.when(t == 0)
    def _():
        pad2_ref[0:1, :, :] = jnp.zeros((1, WH + 2, C), jnp.float32)

    @pl.when(t == nt - 1)
    def _():
        pad2_ref[HH + 1:HH + 2, :, :] = jnp.zeros((1, WH + 2, C), jnp.float32)

    taps2 = [pad2_ref[dy:dy + HH, dx:dx + WH, :].astype(jnp.bfloat16)
             for dy in range(3) for dx in range(3)]
    patches2 = jnp.concatenate(taps2, axis=-1).reshape(HH * WH, 9 * C)

    y = jnp.dot(patches2, w2_ref[...],
                preferred_element_type=jnp.float32) + b2_ref[...]

    norm = jnp.dot((y * y).astype(jnp.bfloat16), g_ref[...],
                   preferred_element_type=jnp.float32) + beta_ref[...]
    out = y * jnp.sqrt(norm)
    out = out + id_ref[...].reshape(HH * WH, C)

    o_ref[...] = out.reshape(1, HH, WH, C)


def kernel(x_nhwc, subpel_w, subpel_b, conv_w, conv_b, gamma, beta, up_w, up_b):
    N, H, W, Cin = x_nhwc.shape
    C_main = subpel_w.shape[-1]
    C_tot = 2 * C_main
    C = conv_w.shape[-1]
    r = 2

    c = jnp.arange(C_main // (r * r))
    perm = jnp.concatenate(
        [jnp.concatenate([c * 4 + p, C_main + c * 4 + p]) for p in range(4)])
    w_cat = jnp.concatenate([subpel_w.reshape(9 * Cin, C_main),
                             up_w.reshape(9 * Cin, C_main)], axis=1)
    w_cat = w_cat[:, perm].astype(jnp.bfloat16)
    b_cat = jnp.concatenate([subpel_b, up_b])[perm].reshape(1, C_tot)
    slope = jnp.where(perm < C_main, 0.01, 1.0).reshape(1, C_tot)

    w2 = conv_w.reshape(9 * C, C).astype(jnp.bfloat16)
    b2 = conv_b.reshape(1, C)
    g2 = gamma.astype(jnp.bfloat16)
    beta2 = beta.reshape(1, C)

    lth = 32 if H % 32 == 0 else H
    nt = H // lth
    HH = 2 * lth
    body = functools.partial(_fused_kernel, lth=lth, H=H, W=W, Cin=Cin,
                             C_tot=C_tot, C=C)
    out = pl.pallas_call(
        body,
        out_shape=jax.ShapeDtypeStruct((N, r * H, r * W, C), jnp.float32),
        grid=(N, nt),
        in_specs=[
            pl.BlockSpec((1, H, W, Cin), lambda n, t: (n, 0, 0, 0)),
            pl.BlockSpec((9 * Cin, C_tot), lambda n, t: (0, 0)),
            pl.BlockSpec((1, C_tot), lambda n, t: (0, 0)),
            pl.BlockSpec((1, C_tot), lambda n, t: (0, 0)),
            pl.BlockSpec((9 * C, C), lambda n, t: (0, 0)),
            pl.BlockSpec((1, C), lambda n, t: (0, 0)),
            pl.BlockSpec((C, C), lambda n, t: (0, 0)),
            pl.BlockSpec((1, C), lambda n, t: (0, 0)),
        ],
        out_specs=pl.BlockSpec((1, HH, r * W, C), lambda n, t: (n, t, 0, 0)),
        scratch_shapes=[
            pltpu.VMEM((lth + 4, W + 2, Cin), jnp.bfloat16),
            pltpu.VMEM((HH + 2, r * W + 2, C), jnp.float32),
            pltpu.VMEM((HH, r * W, C), jnp.float32),
        ],
        compiler_params=pltpu.CompilerParams(
            dimension_semantics=("parallel", "parallel")),
    )(x_nhwc, w_cat, b_cat, slope, w2, b2, g2, beta2)

    return out
